# baseline (speedup 1.0000x reference)
import functools

import jax
import jax.numpy as jnp
from jax import lax
from jax.experimental import pallas as pl
from jax.experimental.pallas import tpu as pltpu
from jax.experimental.pallas import tpu_sc as plsc

_CHUNK = 80


def _acc_slices(n_nodes, ns):
  per = (n_nodes // ns) & ~7
  tail0 = per * ns
  tail = n_nodes - tail0
  return per, tail0, tail


def _pipelined_agg(n_iter, base_blk, packed_hbm, table_hbm, acc_sh,
                   idx, rows, gs, ss):
  def fetch(i, k):
    pltpu.sync_copy(packed_hbm.at[base_blk + i], idx[k])

  def start_g(k):
    pltpu.async_copy(table_hbm.at[idx[k].at[0]], rows[k], gs[k])

  def wait_g(k):
    pltpu.make_async_copy(table_hbm.at[idx[k].at[0]], rows[k], gs[k]).wait()

  def start_s(k):
    pltpu.async_copy(rows[k], acc_sh.at[idx[k].at[1]], ss[k], add=True)

  def wait_s(k):
    pltpu.make_async_copy(rows[k], acc_sh.at[idx[k].at[1]], ss[k]).wait()

  def substep(i, k, q=None):
    kn = (k + 2) % 4
    if q is None:
      if i >= 2:
        wait_s(kn)
      if i + 2 < n_iter:
        fetch(i + 2, kn)
        start_g(kn)
    else:
      @pl.when(i >= 2)
      def _():
        wait_s(kn)

      @pl.when(i + 2 < n_iter)
      def _():
        fetch(i + 2, kn)
        start_g(kn)

    wait_g(k)
    start_s(k)

  fetch(0, 0)
  start_g(0)
  fetch(1, 1)
  start_g(1)

  nq = n_iter // 4

  @pl.loop(0, nq)
  def _(q):
    i0 = q * 4
    for k in range(4):
      substep(i0 + k, k, q=q)

  for i in range(nq * 4, n_iter):
    substep(i, i % 4)

  wait_s((n_iter - 2) % 4)
  wait_s((n_iter - 1) % 4)


def _make_agg_edge_split(n_nodes, n_acc, n_edges, feat):
  info = plsc.get_sparse_core_info()
  nc, ns = info.num_cores, info.num_subcores
  per_tile = n_edges // (nc * ns)
  n_iter = per_tile // _CHUNK
  per, tail0, tail = _acc_slices(n_nodes, ns)
  mesh = plsc.VectorSubcoreMesh(core_axis_name="c", subcore_axis_name="s")

  @functools.partial(
      pl.kernel,
      mesh=mesh,
      out_type=[
          jax.ShapeDtypeStruct((n_nodes, feat), jnp.float32),
          jax.ShapeDtypeStruct((n_nodes, feat), jnp.float32),
      ],
      scratch_types=[
          pltpu.VMEM((2, _CHUNK), jnp.int32),
          pltpu.VMEM((2, _CHUNK), jnp.int32),
          pltpu.VMEM((2, _CHUNK), jnp.int32),
          pltpu.VMEM((2, _CHUNK), jnp.int32),
          pltpu.VMEM((_CHUNK, feat), jnp.float32),
          pltpu.VMEM((_CHUNK, feat), jnp.float32),
          pltpu.VMEM((_CHUNK, feat), jnp.float32),
          pltpu.VMEM((_CHUNK, feat), jnp.float32),
          pltpu.VMEM_SHARED((n_acc, feat), jnp.float32),
          pltpu.SemaphoreType.DMA,
          pltpu.SemaphoreType.DMA,
          pltpu.SemaphoreType.DMA,
          pltpu.SemaphoreType.DMA,
          pltpu.SemaphoreType.DMA,
          pltpu.SemaphoreType.DMA,
          pltpu.SemaphoreType.DMA,
          pltpu.SemaphoreType.DMA,
      ],
  )
  def agg(packed_hbm, table_hbm, zeros_hbm, out0_hbm, out1_hbm,
          i0, i1, i2, i3, r0, r1, r2, r3, acc_sh,
          g0, g1, g2, g3, s0, s1, s2, s3):
    c = lax.axis_index("c")
    s = lax.axis_index("s")
    w = c * ns + s
    sl = pl.ds(pl.multiple_of(s * per, 8), per)
    pltpu.sync_copy(zeros_hbm.at[sl], acc_sh.at[sl])

    @pl.when(s == ns - 1)
    def _():
      tl = pl.ds(tail0, tail)
      pltpu.sync_copy(zeros_hbm.at[tl], acc_sh.at[tl])

    plsc.subcore_barrier()
    _pipelined_agg(n_iter, w * n_iter, packed_hbm, table_hbm, acc_sh,
                   (i0, i1, i2, i3), (r0, r1, r2, r3),
                   (g0, g1, g2, g3), (s0, s1, s2, s3))
    plsc.subcore_barrier()

    def write_out(out_hbm):
      pltpu.sync_copy(acc_sh.at[sl], out_hbm.at[sl])

      @pl.when(s == ns - 1)
      def _():
        tl = pl.ds(tail0, tail)
        pltpu.sync_copy(acc_sh.at[tl], out_hbm.at[tl])

    @pl.when(c == 0)
    def _():
      write_out(out0_hbm)

    @pl.when(c == 1)
    def _():
      write_out(out1_hbm)

  return agg


def _make_agg_feat_split(n_nodes, n_acc, n_edges, feat_half):
  info = plsc.get_sparse_core_info()
  ns = info.num_subcores
  per_tile = n_edges // ns
  n_iter = per_tile // _CHUNK
  per, tail0, tail = _acc_slices(n_nodes, ns)
  mesh = plsc.VectorSubcoreMesh(core_axis_name="c", subcore_axis_name="s")

  @functools.partial(
      pl.kernel,
      mesh=mesh,
      out_type=[
          jax.ShapeDtypeStruct((n_nodes, feat_half), jnp.float32),
          jax.ShapeDtypeStruct((n_nodes, feat_half), jnp.float32),
      ],
      scratch_types=[
          pltpu.VMEM((2, _CHUNK), jnp.int32),
          pltpu.VMEM((2, _CHUNK), jnp.int32),
          pltpu.VMEM((2, _CHUNK), jnp.int32),
          pltpu.VMEM((2, _CHUNK), jnp.int32),
          pltpu.VMEM((_CHUNK, feat_half), jnp.float32),
          pltpu.VMEM((_CHUNK, feat_half), jnp.float32),
          pltpu.VMEM((_CHUNK, feat_half), jnp.float32),
          pltpu.VMEM((_CHUNK, feat_half), jnp.float32),
          pltpu.VMEM_SHARED((n_acc, feat_half), jnp.float32),
          pltpu.SemaphoreType.DMA,
          pltpu.SemaphoreType.DMA,
          pltpu.SemaphoreType.DMA,
          pltpu.SemaphoreType.DMA,
          pltpu.SemaphoreType.DMA,
          pltpu.SemaphoreType.DMA,
          pltpu.SemaphoreType.DMA,
          pltpu.SemaphoreType.DMA,
      ],
  )
  def agg(packed_hbm, t0_hbm, t1_hbm, zeros_hbm, out0_hbm, out1_hbm,
          i0, i1, i2, i3, r0, r1, r2, r3, acc_sh,
          g0, g1, g2, g3, s0, s1, s2, s3):
    c = lax.axis_index("c")
    s = lax.axis_index("s")
    sl = pl.ds(pl.multiple_of(s * per, 8), per)
    pltpu.sync_copy(zeros_hbm.at[sl], acc_sh.at[sl])

    @pl.when(s == ns - 1)
    def _():
      tl = pl.ds(tail0, tail)
      pltpu.sync_copy(zeros_hbm.at[tl], acc_sh.at[tl])

    plsc.subcore_barrier()

    @pl.when(c == 0)
    def _():
      _pipelined_agg(n_iter, s * n_iter, packed_hbm, t0_hbm, acc_sh,
                     (i0, i1, i2, i3), (r0, r1, r2, r3),
                     (g0, g1, g2, g3), (s0, s1, s2, s3))

    @pl.when(c == 1)
    def _():
      _pipelined_agg(n_iter, s * n_iter, packed_hbm, t1_hbm, acc_sh,
                     (i0, i1, i2, i3), (r0, r1, r2, r3),
                     (g0, g1, g2, g3), (s0, s1, s2, s3))

    plsc.subcore_barrier()

    def write_out(out_hbm):
      pltpu.sync_copy(acc_sh.at[sl], out_hbm.at[sl])

      @pl.when(s == ns - 1)
      def _():
        tl = pl.ds(tail0, tail)
        pltpu.sync_copy(acc_sh.at[tl], out_hbm.at[tl])

    @pl.when(c == 0)
    def _():
      write_out(out0_hbm)

    @pl.when(c == 1)
    def _():
      write_out(out1_hbm)

  return agg


_BLK = 1000


def _norm_body(x_ref, o_ref):
  x = x_ref[...]
  s = jnp.sum(x * x, axis=1, keepdims=True)
  o_ref[...] = x / jnp.maximum(jnp.sqrt(s), 1e-12)


def _tc_norm(x):
  n, f = x.shape
  grid = n // _BLK
  return pl.pallas_call(
      _norm_body,
      grid=(grid,),
      in_specs=[pl.BlockSpec((_BLK, f), lambda i: (i, 0))],
      out_specs=pl.BlockSpec((_BLK, f), lambda i: (i, 0)),
      out_shape=jax.ShapeDtypeStruct((n, f), jnp.float32),
  )(x)


def _layer1_body(a0_ref, a1_ref, w_ref, b_ref, o0_ref, o1_ref):
  a = a0_ref[...] + a1_ref[...]
  h = jnp.dot(a.astype(jnp.bfloat16), w_ref[...].astype(jnp.bfloat16),
              preferred_element_type=jnp.float32) + b_ref[...]
  h = jnp.maximum(h, 0.0)
  s = jnp.sum(h * h, axis=1, keepdims=True)
  y = h / jnp.maximum(jnp.sqrt(s), 1e-12)
  hh = y.shape[1] // 2
  o0_ref[...] = y[:, :hh]
  o1_ref[...] = y[:, hh:]


def _tc_layer1(p0, p1, w, b):
  n, f = p0.shape
  h = w.shape[1]
  grid = n // _BLK
  return pl.pallas_call(
      _layer1_body,
      grid=(grid,),
      in_specs=[
          pl.BlockSpec((_BLK, f), lambda i: (i, 0)),
          pl.BlockSpec((_BLK, f), lambda i: (i, 0)),
          pl.BlockSpec((f, h), lambda i: (0, 0)),
          pl.BlockSpec((1, h), lambda i: (0, 0)),
      ],
      out_specs=[
          pl.BlockSpec((_BLK, h // 2), lambda i: (i, 0)),
          pl.BlockSpec((_BLK, h // 2), lambda i: (i, 0)),
      ],
      out_shape=[
          jax.ShapeDtypeStruct((n, h // 2), jnp.float32),
          jax.ShapeDtypeStruct((n, h // 2), jnp.float32),
      ],
  )(p0, p1, w, b.reshape(1, h))


def _mid_body(a0_ref, a1_ref, w_ref, b_ref, o0_ref, o1_ref):
  a = jnp.concatenate([a0_ref[...], a1_ref[...]], axis=1)
  h = jnp.dot(a.astype(jnp.bfloat16), w_ref[...].astype(jnp.bfloat16),
              preferred_element_type=jnp.float32) + b_ref[...]
  h = jnp.maximum(h, 0.0)
  s = jnp.sum(h * h, axis=1, keepdims=True)
  y = h / jnp.maximum(jnp.sqrt(s), 1e-12)
  hh = y.shape[1] // 2
  o0_ref[...] = y[:, :hh]
  o1_ref[...] = y[:, hh:]


def _tc_mid_layer(a0, a1, w, b):
  n, fh = a0.shape
  f = 2 * fh
  h = w.shape[1]
  grid = n // _BLK
  return pl.pallas_call(
      _mid_body,
      grid=(grid,),
      in_specs=[
          pl.BlockSpec((_BLK, fh), lambda i: (i, 0)),
          pl.BlockSpec((_BLK, fh), lambda i: (i, 0)),
          pl.BlockSpec((f, h), lambda i: (0, 0)),
          pl.BlockSpec((1, h), lambda i: (0, 0)),
      ],
      out_specs=[
          pl.BlockSpec((_BLK, h // 2), lambda i: (i, 0)),
          pl.BlockSpec((_BLK, h // 2), lambda i: (i, 0)),
      ],
      out_shape=[
          jax.ShapeDtypeStruct((n, h // 2), jnp.float32),
          jax.ShapeDtypeStruct((n, h // 2), jnp.float32),
      ],
  )(a0, a1, w, b.reshape(1, h))


def _final_body(a0_ref, a1_ref, w_ref, b_ref, w1_ref, b1_ref, w2_ref, b2_ref,
                o_ref, acc_ref, *, inv_n):
  i = pl.program_id(0)

  @pl.when(i == 0)
  def _():
    acc_ref[...] = jnp.zeros_like(acc_ref)

  a = jnp.concatenate([a0_ref[...], a1_ref[...]], axis=1)
  hm = jnp.dot(a.astype(jnp.bfloat16), w_ref[...].astype(jnp.bfloat16),
               preferred_element_type=jnp.float32) + b_ref[...]
  hm = jnp.maximum(hm, 0.0)
  acc_ref[...] += jnp.sum(hm, axis=0, keepdims=True)

  @pl.when(i == pl.num_programs(0) - 1)
  def _():
    g = acc_ref[...] * inv_n
    g = jnp.dot(g, w1_ref[...],
                preferred_element_type=jnp.float32) + b1_ref[...]
    g = jnp.maximum(g, 0.0)
    z = jnp.dot(g, w2_ref[...],
                preferred_element_type=jnp.float32) + b2_ref[...]
    m = jnp.max(z, axis=1, keepdims=True)
    e = jnp.exp(z - m)
    lse = jnp.log(jnp.sum(e, axis=1, keepdims=True)) + m
    o_ref[...] = z - lse


def _tc_final_head(a0, a1, w, b, w1, b1, w2, b2, n_nodes):
  n, fh = a0.shape
  f = 2 * fh
  h = w.shape[1]
  c = w2.shape[1]
  grid = n // _BLK
  return pl.pallas_call(
      functools.partial(_final_body, inv_n=1.0 / n_nodes),
      grid=(grid,),
      in_specs=[
          pl.BlockSpec((_BLK, fh), lambda i: (i, 0)),
          pl.BlockSpec((_BLK, fh), lambda i: (i, 0)),
          pl.BlockSpec((f, h), lambda i: (0, 0)),
          pl.BlockSpec((1, h), lambda i: (0, 0)),
          pl.BlockSpec((h, h), lambda i: (0, 0)),
          pl.BlockSpec((1, h), lambda i: (0, 0)),
          pl.BlockSpec((h, c), lambda i: (0, 0)),
          pl.BlockSpec((1, c), lambda i: (0, 0)),
      ],
      out_specs=pl.BlockSpec((1, c), lambda i: (0, 0)),
      out_shape=jax.ShapeDtypeStruct((1, c), jnp.float32),
      scratch_shapes=[pltpu.VMEM((1, h), jnp.float32)],
  )(a0, a1, w, b.reshape(1, h), w1, b1.reshape(1, h), w2, b2.reshape(1, c))


def kernel(x, edge_index, batch, W1, b1, Wc1, bc1, Wc2, bc2,
           lin1_W, lin1_b, lin2_W, lin2_b):
  n, f_in = x.shape
  e = edge_index.shape[1]
  h = W1.shape[1]
  del batch

  info = plsc.get_sparse_core_info()
  blk = info.num_cores * info.num_subcores * _CHUNK
  e_pad = ((e + blk - 1) // blk) * blk
  pad = e_pad - e
  if pad:
    src_p = jnp.concatenate([edge_index[0], jnp.zeros((pad,), jnp.int32)])
    dst_p = jnp.concatenate([edge_index[1], jnp.full((pad,), n, jnp.int32)])
  else:
    src_p, dst_p = edge_index[0], edge_index[1]
  packed = jnp.stack(
      [src_p.reshape(e_pad // _CHUNK, _CHUNK),
       dst_p.reshape(e_pad // _CHUNK, _CHUNK)], axis=1)
  n_acc = n + 16
  zeros_f = jnp.zeros((n, h // 2), jnp.float32)

  agg1 = _make_agg_edge_split(n, n_acc, e_pad, f_in)
  aggf = _make_agg_feat_split(n, n_acc, e_pad, h // 2)

  xn = _tc_norm(x)
  p0, p1 = agg1(packed, xn, zeros_f[:, :f_in])
  h0, h1 = _tc_layer1(p0, p1, W1, b1)
  a0, a1 = aggf(packed, h0, h1, zeros_f)
  h0, h1 = _tc_mid_layer(a0, a1, Wc1, bc1)
  a0, a1 = aggf(packed, h0, h1, zeros_f)
  return _tc_final_head(a0, a1, Wc2, bc2, lin1_W, lin1_b, lin2_W, lin2_b, n)


# speedup vs baseline: 1.0013x; 1.0013x over previous
import functools

import jax
import jax.numpy as jnp
from jax import lax
from jax.experimental import pallas as pl
from jax.experimental.pallas import tpu as pltpu
from jax.experimental.pallas import tpu_sc as plsc

_CHUNK = 80


def _acc_slices(n_nodes, ns):
  per = (n_nodes // ns) & ~7
  tail0 = per * ns
  tail = n_nodes - tail0
  return per, tail0, tail


def _pipelined_agg(n_iter, base_blk, packed_hbm, table_hbm, acc_sh,
                   idx, rows, gs, ss):
  def fetch(i, k):
    pltpu.sync_copy(packed_hbm.at[base_blk + i], idx[k])

  def start_g(k):
    pltpu.async_copy(table_hbm.at[idx[k].at[0]], rows[k], gs[k])

  def wait_g(k):
    pltpu.make_async_copy(table_hbm.at[idx[k].at[0]], rows[k], gs[k]).wait()

  def start_s(k):
    pltpu.async_copy(rows[k], acc_sh.at[idx[k].at[1]], ss[k], add=True)

  def wait_s(k):
    pltpu.make_async_copy(rows[k], acc_sh.at[idx[k].at[1]], ss[k]).wait()

  def substep(i, k, q=None):
    kn = (k + 2) % 4
    if q is None:
      if i >= 2:
        wait_s(kn)
      if i + 2 < n_iter:
        fetch(i + 2, kn)
        start_g(kn)
    else:
      @pl.when(i >= 2)
      def _():
        wait_s(kn)

      @pl.when(i + 2 < n_iter)
      def _():
        fetch(i + 2, kn)
        start_g(kn)

    wait_g(k)
    start_s(k)

  fetch(0, 0)
  start_g(0)
  fetch(1, 1)
  start_g(1)
  plsc.subcore_barrier()

  nq = n_iter // 4

  @pl.loop(0, nq)
  def _(q):
    i0 = q * 4
    for k in range(4):
      substep(i0 + k, k, q=q)

  for i in range(nq * 4, n_iter):
    substep(i, i % 4)

  wait_s((n_iter - 2) % 4)
  wait_s((n_iter - 1) % 4)


def _make_agg_edge_split(n_nodes, n_acc, n_edges, feat):
  info = plsc.get_sparse_core_info()
  nc, ns = info.num_cores, info.num_subcores
  per_tile = n_edges // (nc * ns)
  n_iter = per_tile // _CHUNK
  per, tail0, tail = _acc_slices(n_nodes, ns)
  mesh = plsc.VectorSubcoreMesh(core_axis_name="c", subcore_axis_name="s")

  @functools.partial(
      pl.kernel,
      mesh=mesh,
      out_type=[
          jax.ShapeDtypeStruct((n_nodes, feat), jnp.float32),
          jax.ShapeDtypeStruct((n_nodes, feat), jnp.float32),
      ],
      scratch_types=[
          pltpu.VMEM((2, _CHUNK), jnp.int32),
          pltpu.VMEM((2, _CHUNK), jnp.int32),
          pltpu.VMEM((2, _CHUNK), jnp.int32),
          pltpu.VMEM((2, _CHUNK), jnp.int32),
          pltpu.VMEM((_CHUNK, feat), jnp.float32),
          pltpu.VMEM((_CHUNK, feat), jnp.float32),
          pltpu.VMEM((_CHUNK, feat), jnp.float32),
          pltpu.VMEM((_CHUNK, feat), jnp.float32),
          pltpu.VMEM_SHARED((n_acc, feat), jnp.float32),
          pltpu.SemaphoreType.DMA,
          pltpu.SemaphoreType.DMA,
          pltpu.SemaphoreType.DMA,
          pltpu.SemaphoreType.DMA,
          pltpu.SemaphoreType.DMA,
          pltpu.SemaphoreType.DMA,
          pltpu.SemaphoreType.DMA,
          pltpu.SemaphoreType.DMA,
      ],
  )
  def agg(packed_hbm, table_hbm, zeros_hbm, out0_hbm, out1_hbm,
          i0, i1, i2, i3, r0, r1, r2, r3, acc_sh,
          g0, g1, g2, g3, s0, s1, s2, s3):
    c = lax.axis_index("c")
    s = lax.axis_index("s")
    w = c * ns + s
    sl = pl.ds(pl.multiple_of(s * per, 8), per)
    pltpu.sync_copy(zeros_hbm.at[sl], acc_sh.at[sl])

    @pl.when(s == ns - 1)
    def _():
      tl = pl.ds(tail0, tail)
      pltpu.sync_copy(zeros_hbm.at[tl], acc_sh.at[tl])

    _pipelined_agg(n_iter, w * n_iter, packed_hbm, table_hbm, acc_sh,
                   (i0, i1, i2, i3), (r0, r1, r2, r3),
                   (g0, g1, g2, g3), (s0, s1, s2, s3))
    plsc.subcore_barrier()

    def write_out(out_hbm):
      pltpu.sync_copy(acc_sh.at[sl], out_hbm.at[sl])

      @pl.when(s == ns - 1)
      def _():
        tl = pl.ds(tail0, tail)
        pltpu.sync_copy(acc_sh.at[tl], out_hbm.at[tl])

    @pl.when(c == 0)
    def _():
      write_out(out0_hbm)

    @pl.when(c == 1)
    def _():
      write_out(out1_hbm)

  return agg


def _make_agg_feat_split(n_nodes, n_acc, n_edges, feat_half):
  info = plsc.get_sparse_core_info()
  ns = info.num_subcores
  per_tile = n_edges // ns
  n_iter = per_tile // _CHUNK
  per, tail0, tail = _acc_slices(n_nodes, ns)
  mesh = plsc.VectorSubcoreMesh(core_axis_name="c", subcore_axis_name="s")

  @functools.partial(
      pl.kernel,
      mesh=mesh,
      out_type=[
          jax.ShapeDtypeStruct((n_nodes, feat_half), jnp.float32),
          jax.ShapeDtypeStruct((n_nodes, feat_half), jnp.float32),
      ],
      scratch_types=[
          pltpu.VMEM((2, _CHUNK), jnp.int32),
          pltpu.VMEM((2, _CHUNK), jnp.int32),
          pltpu.VMEM((2, _CHUNK), jnp.int32),
          pltpu.VMEM((2, _CHUNK), jnp.int32),
          pltpu.VMEM((_CHUNK, feat_half), jnp.float32),
          pltpu.VMEM((_CHUNK, feat_half), jnp.float32),
          pltpu.VMEM((_CHUNK, feat_half), jnp.float32),
          pltpu.VMEM((_CHUNK, feat_half), jnp.float32),
          pltpu.VMEM_SHARED((n_acc, feat_half), jnp.float32),
          pltpu.SemaphoreType.DMA,
          pltpu.SemaphoreType.DMA,
          pltpu.SemaphoreType.DMA,
          pltpu.SemaphoreType.DMA,
          pltpu.SemaphoreType.DMA,
          pltpu.SemaphoreType.DMA,
          pltpu.SemaphoreType.DMA,
          pltpu.SemaphoreType.DMA,
      ],
  )
  def agg(packed_hbm, t0_hbm, t1_hbm, zeros_hbm, out0_hbm, out1_hbm,
          i0, i1, i2, i3, r0, r1, r2, r3, acc_sh,
          g0, g1, g2, g3, s0, s1, s2, s3):
    c = lax.axis_index("c")
    s = lax.axis_index("s")
    sl = pl.ds(pl.multiple_of(s * per, 8), per)
    pltpu.sync_copy(zeros_hbm.at[sl], acc_sh.at[sl])

    @pl.when(s == ns - 1)
    def _():
      tl = pl.ds(tail0, tail)
      pltpu.sync_copy(zeros_hbm.at[tl], acc_sh.at[tl])

    @pl.when(c == 0)
    def _():
      _pipelined_agg(n_iter, s * n_iter, packed_hbm, t0_hbm, acc_sh,
                     (i0, i1, i2, i3), (r0, r1, r2, r3),
                     (g0, g1, g2, g3), (s0, s1, s2, s3))

    @pl.when(c == 1)
    def _():
      _pipelined_agg(n_iter, s * n_iter, packed_hbm, t1_hbm, acc_sh,
                     (i0, i1, i2, i3), (r0, r1, r2, r3),
                     (g0, g1, g2, g3), (s0, s1, s2, s3))

    plsc.subcore_barrier()

    def write_out(out_hbm):
      pltpu.sync_copy(acc_sh.at[sl], out_hbm.at[sl])

      @pl.when(s == ns - 1)
      def _():
        tl = pl.ds(tail0, tail)
        pltpu.sync_copy(acc_sh.at[tl], out_hbm.at[tl])

    @pl.when(c == 0)
    def _():
      write_out(out0_hbm)

    @pl.when(c == 1)
    def _():
      write_out(out1_hbm)

  return agg


_BLK = 1000


def _norm_body(x_ref, o_ref):
  x = x_ref[...]
  s = jnp.sum(x * x, axis=1, keepdims=True)
  o_ref[...] = x / jnp.maximum(jnp.sqrt(s), 1e-12)


def _tc_norm(x):
  n, f = x.shape
  grid = n // _BLK
  return pl.pallas_call(
      _norm_body,
      grid=(grid,),
      in_specs=[pl.BlockSpec((_BLK, f), lambda i: (i, 0))],
      out_specs=pl.BlockSpec((_BLK, f), lambda i: (i, 0)),
      out_shape=jax.ShapeDtypeStruct((n, f), jnp.float32),
  )(x)


def _layer1_body(a0_ref, a1_ref, w_ref, b_ref, o0_ref, o1_ref):
  a = a0_ref[...] + a1_ref[...]
  h = jnp.dot(a, w_ref[...], preferred_element_type=jnp.float32) + b_ref[...]
  h = jnp.maximum(h, 0.0)
  s = jnp.sum(h * h, axis=1, keepdims=True)
  y = h / jnp.maximum(jnp.sqrt(s), 1e-12)
  hh = y.shape[1] // 2
  o0_ref[...] = y[:, :hh]
  o1_ref[...] = y[:, hh:]


def _tc_layer1(p0, p1, w, b):
  n, f = p0.shape
  h = w.shape[1]
  grid = n // _BLK
  return pl.pallas_call(
      _layer1_body,
      grid=(grid,),
      in_specs=[
          pl.BlockSpec((_BLK, f), lambda i: (i, 0)),
          pl.BlockSpec((_BLK, f), lambda i: (i, 0)),
          pl.BlockSpec((f, h), lambda i: (0, 0)),
          pl.BlockSpec((1, h), lambda i: (0, 0)),
      ],
      out_specs=[
          pl.BlockSpec((_BLK, h // 2), lambda i: (i, 0)),
          pl.BlockSpec((_BLK, h // 2), lambda i: (i, 0)),
      ],
      out_shape=[
          jax.ShapeDtypeStruct((n, h // 2), jnp.float32),
          jax.ShapeDtypeStruct((n, h // 2), jnp.float32),
      ],
  )(p0, p1, w, b.reshape(1, h))


def _mid_body(a0_ref, a1_ref, w_ref, b_ref, o0_ref, o1_ref):
  a = jnp.concatenate([a0_ref[...], a1_ref[...]], axis=1)
  h = jnp.dot(a, w_ref[...], preferred_element_type=jnp.float32) + b_ref[...]
  h = jnp.maximum(h, 0.0)
  s = jnp.sum(h * h, axis=1, keepdims=True)
  y = h / jnp.maximum(jnp.sqrt(s), 1e-12)
  hh = y.shape[1] // 2
  o0_ref[...] = y[:, :hh]
  o1_ref[...] = y[:, hh:]


def _tc_mid_layer(a0, a1, w, b):
  n, fh = a0.shape
  f = 2 * fh
  h = w.shape[1]
  grid = n // _BLK
  return pl.pallas_call(
      _mid_body,
      grid=(grid,),
      in_specs=[
          pl.BlockSpec((_BLK, fh), lambda i: (i, 0)),
          pl.BlockSpec((_BLK, fh), lambda i: (i, 0)),
          pl.BlockSpec((f, h), lambda i: (0, 0)),
          pl.BlockSpec((1, h), lambda i: (0, 0)),
      ],
      out_specs=[
          pl.BlockSpec((_BLK, h // 2), lambda i: (i, 0)),
          pl.BlockSpec((_BLK, h // 2), lambda i: (i, 0)),
      ],
      out_shape=[
          jax.ShapeDtypeStruct((n, h // 2), jnp.float32),
          jax.ShapeDtypeStruct((n, h // 2), jnp.float32),
      ],
  )(a0, a1, w, b.reshape(1, h))


def _final_body(a0_ref, a1_ref, w_ref, b_ref, w1_ref, b1_ref, w2_ref, b2_ref,
                o_ref, acc_ref, *, inv_n):
  i = pl.program_id(0)

  @pl.when(i == 0)
  def _():
    acc_ref[...] = jnp.zeros_like(acc_ref)

  a = jnp.concatenate([a0_ref[...], a1_ref[...]], axis=1)
  hm = jnp.dot(a, w_ref[...], preferred_element_type=jnp.float32) + b_ref[...]
  hm = jnp.maximum(hm, 0.0)
  acc_ref[...] += jnp.sum(hm, axis=0, keepdims=True)

  @pl.when(i == pl.num_programs(0) - 1)
  def _():
    g = acc_ref[...] * inv_n
    g = jnp.dot(g, w1_ref[...],
                preferred_element_type=jnp.float32) + b1_ref[...]
    g = jnp.maximum(g, 0.0)
    z = jnp.dot(g, w2_ref[...],
                preferred_element_type=jnp.float32) + b2_ref[...]
    m = jnp.max(z, axis=1, keepdims=True)
    e = jnp.exp(z - m)
    lse = jnp.log(jnp.sum(e, axis=1, keepdims=True)) + m
    o_ref[...] = z - lse


def _tc_final_head(a0, a1, w, b, w1, b1, w2, b2, n_nodes):
  n, fh = a0.shape
  f = 2 * fh
  h = w.shape[1]
  c = w2.shape[1]
  grid = n // _BLK
  return pl.pallas_call(
      functools.partial(_final_body, inv_n=1.0 / n_nodes),
      grid=(grid,),
      in_specs=[
          pl.BlockSpec((_BLK, fh), lambda i: (i, 0)),
          pl.BlockSpec((_BLK, fh), lambda i: (i, 0)),
          pl.BlockSpec((f, h), lambda i: (0, 0)),
          pl.BlockSpec((1, h), lambda i: (0, 0)),
          pl.BlockSpec((h, h), lambda i: (0, 0)),
          pl.BlockSpec((1, h), lambda i: (0, 0)),
          pl.BlockSpec((h, c), lambda i: (0, 0)),
          pl.BlockSpec((1, c), lambda i: (0, 0)),
      ],
      out_specs=pl.BlockSpec((1, c), lambda i: (0, 0)),
      out_shape=jax.ShapeDtypeStruct((1, c), jnp.float32),
      scratch_shapes=[pltpu.VMEM((1, h), jnp.float32)],
  )(a0, a1, w, b.reshape(1, h), w1, b1.reshape(1, h), w2, b2.reshape(1, c))


def kernel(x, edge_index, batch, W1, b1, Wc1, bc1, Wc2, bc2,
           lin1_W, lin1_b, lin2_W, lin2_b):
  n, f_in = x.shape
  e = edge_index.shape[1]
  h = W1.shape[1]
  del batch

  info = plsc.get_sparse_core_info()
  blk = info.num_cores * info.num_subcores * _CHUNK
  e_pad = ((e + blk - 1) // blk) * blk
  pad = e_pad - e
  if pad:
    src_p = jnp.concatenate([edge_index[0], jnp.zeros((pad,), jnp.int32)])
    dst_p = jnp.concatenate([edge_index[1], jnp.full((pad,), n, jnp.int32)])
  else:
    src_p, dst_p = edge_index[0], edge_index[1]
  packed = jnp.stack(
      [src_p.reshape(e_pad // _CHUNK, _CHUNK),
       dst_p.reshape(e_pad // _CHUNK, _CHUNK)], axis=1)
  n_acc = n + 16
  zeros_f = jnp.zeros((n, h // 2), jnp.float32)

  agg1 = _make_agg_edge_split(n, n_acc, e_pad, f_in)
  aggf = _make_agg_feat_split(n, n_acc, e_pad, h // 2)

  xn = _tc_norm(x)
  p0, p1 = agg1(packed, xn, zeros_f[:, :f_in])
  h0, h1 = _tc_layer1(p0, p1, W1, b1)
  a0, a1 = aggf(packed, h0, h1, zeros_f)
  h0, h1 = _tc_mid_layer(a0, a1, Wc1, bc1)
  a0, a1 = aggf(packed, h0, h1, zeros_f)
  return _tc_final_head(a0, a1, Wc2, bc2, lin1_W, lin1_b, lin2_W, lin2_b, n)
